# Initial kernel scaffold; baseline (speedup 1.0000x reference)
import jax
import jax.numpy as jnp
from jax.experimental import pallas as pl


def kernel(x, emb, W1, b1):
    raise NotImplementedError("write your pallas kernel here")


# speedup vs baseline: 2.7789x; 2.7789x over previous
import functools

import jax
import jax.numpy as jnp
from jax import lax
from jax.experimental import pallas as pl
from jax.experimental.pallas import tpu as pltpu
from jax.experimental.pallas import tpu_sc as plsc

_VOCAB = 100000
_D = 128
_L = 200
_B = 4096
_H = 7
_HP = 8
_TOK = _B * _L
_NW = 32
_PER_W = _TOK // _NW
_CHUNK = 6400
_NCHUNK = _PER_W // _CHUNK


def _mm_body(emb_ref, w_ref, t_ref):
    t_ref[...] = jnp.dot(emb_ref[...], w_ref[...],
                         preferred_element_type=jnp.float32)


def _build_t(emb, w1r):
    mb = 1000
    return pl.pallas_call(
        _mm_body,
        grid=(_VOCAB // mb,),
        in_specs=[
            pl.BlockSpec((mb, _D), lambda i: (i, 0)),
            pl.BlockSpec((_D, _L * _HP), lambda i: (0, 0)),
        ],
        out_specs=pl.BlockSpec((mb, _L * _HP), lambda i: (i, 0)),
        out_shape=jax.ShapeDtypeStruct((_VOCAB, _L * _HP), jnp.float32),
    )(emb, w1r)


def _idx_body(x_ref, o_ref):
    o_ref[...] = x_ref[...] * _L + lax.broadcasted_iota(
        jnp.int32, (_B, _L), 1)


def _build_idx(x):
    return pl.pallas_call(
        _idx_body,
        out_shape=jax.ShapeDtypeStruct((_B, _L), jnp.int32),
    )(x)


def _sc_gather_body(t_hbm, idx_hbm, out_hbm, idx_v, rows_v, sem):
    wid = lax.axis_index("s") * 2 + lax.axis_index("c")

    def chunk(r, carry):
        base = wid * _PER_W + r * _CHUNK
        pltpu.sync_copy(idx_hbm.at[pl.ds(base, _CHUNK)], idx_v)
        pltpu.async_copy(t_hbm.at[idx_v], rows_v, sem).wait()
        pltpu.sync_copy(rows_v, out_hbm.at[pl.ds(base, _CHUNK)])
        return carry

    lax.fori_loop(0, _NCHUNK, chunk, 0)


_sc_gather = functools.partial(
    pl.kernel,
    out_type=jax.ShapeDtypeStruct((_TOK, _HP), jnp.float32),
    mesh=plsc.VectorSubcoreMesh(core_axis_name="c", subcore_axis_name="s"),
    scratch_types=[
        pltpu.VMEM((_CHUNK,), jnp.int32),
        pltpu.VMEM((_CHUNK, _HP), jnp.float32),
        pltpu.SemaphoreType.DMA,
    ],
    compiler_params=pltpu.CompilerParams(use_tc_tiling_on_sc=False),
)(_sc_gather_body)


def _red_body(h_ref, s_ref, b_ref, o_ref):
    acc = jnp.dot(h_ref[...], s_ref[...], preferred_element_type=jnp.float32)
    o_ref[...] = jax.nn.sigmoid(acc + b_ref[...])


def _reduce(h2, sel, b1row):
    bb = 512
    return pl.pallas_call(
        _red_body,
        grid=(_B // bb,),
        in_specs=[
            pl.BlockSpec((bb, _L * _HP), lambda i: (i, 0)),
            pl.BlockSpec((_L * _HP, _H), lambda i: (0, 0)),
            pl.BlockSpec((1, _H), lambda i: (0, 0)),
        ],
        out_specs=pl.BlockSpec((bb, _H), lambda i: (i, 0)),
        out_shape=jax.ShapeDtypeStruct((_B, _H), jnp.float32),
    )(h2, sel, b1row)


def kernel(x, emb, W1, b1):
    x = x.astype(jnp.int32)
    w1r = W1.reshape(_L, _D, _H).transpose(1, 0, 2)
    w1r = jnp.pad(w1r, ((0, 0), (0, 0), (0, _HP - _H)))
    w1r = w1r.reshape(_D, _L * _HP)

    t = _build_t(emb, w1r).reshape(_VOCAB * _L, _HP)
    idx = _build_idx(x).reshape(_TOK)
    rows = _sc_gather(t, idx)

    sel = jnp.tile(
        jnp.concatenate([jnp.eye(_H, dtype=jnp.float32),
                         jnp.zeros((_HP - _H, _H), jnp.float32)]), (_L, 1))
    return _reduce(rows.reshape(_B, _L * _HP), sel, b1.reshape(1, _H))
